# Initial kernel scaffold; baseline (speedup 1.0000x reference)
import jax
import jax.numpy as jnp
from jax.experimental import pallas as pl


def kernel(positions, cell, species, r_bins, q_bins, b_coh):
    raise NotImplementedError("write your pallas kernel here")


# speedup vs baseline: 26.3322x; 26.3322x over previous
import functools

import jax
import jax.numpy as jnp
from jax import lax
from jax.experimental import pallas as pl
from jax.experimental.pallas import tpu as pltpu
from jax.experimental.pallas import tpu_sc as plsc

N = 4096
NBINS = 500
NQ = 400
BR = 128
NKEY = 512
NH = 3 * NKEY + 16
DEAD = 3 * NKEY

NC, NS, NL = 2, 16, 16
NW = NC * NS
TOTAL = N * N
CHUNK = TOTAL // NW
SUB = 32768
NSUB = CHUNK // SUB


def _pairs_kernel(row_ref, col_ref, tab_ref, par_ref, keys_ref, qout_ref):
    L = par_ref[0, 0].astype(jnp.bfloat16).astype(jnp.float32)
    r0 = par_ref[0, 1]
    dr = par_ref[0, 2]
    rlast = par_ref[0, 3]
    cutoff = par_ref[0, 4]

    fx_i = row_ref[:, 0:1]
    fy_i = row_ref[:, 1:2]
    fz_i = row_ref[:, 2:3]
    s_i = row_ref[:, 3:4]
    fx_j = col_ref[0:1, :]
    fy_j = col_ref[1:2, :]
    fz_j = col_ref[2:3, :]
    s_j = col_ref[3:4, :]

    def mimg(a, b):
        d = a - b
        d = d - jnp.round(d)
        return d.astype(jnp.bfloat16).astype(jnp.float32)

    dx = mimg(fx_i, fx_j) * L
    dy = mimg(fy_i, fy_j) * L
    dz = mimg(fz_i, fz_j) * L
    r = jnp.sqrt(dx * dx + dy * dy + dz * dz + 1e-12)

    idx = jnp.clip(jnp.floor((r - r0) / dr).astype(jnp.int32), 0, NBINS - 1)
    valid = (r >= r0) & (r < rlast + dr)
    cls = (s_i + s_j).astype(jnp.int32)
    key = idx + cls * NKEY
    keys_ref[...] = jnp.where(valid, key, DEAD)

    row_g = pl.program_id(0) * BR + lax.broadcasted_iota(jnp.int32, (BR, N), 0)
    col_g = lax.broadcasted_iota(jnp.int32, (BR, N), 1)
    eye = row_g == col_g
    cur = r + jnp.where(eye | (s_j < 0.5), 1e6, 0.0)

    ux, uy, uz = [], [], []
    m3 = None
    for _ in range(4):
        mk = jnp.min(cur, axis=1, keepdims=True)
        m3 = mk
        jstar = jnp.min(jnp.where(cur == mk, col_g, N), axis=1, keepdims=True)
        hit = col_g == jstar
        oh = hit.astype(jnp.float32)
        g = jax.lax.dot_general(
            oh, tab_ref[...], (((1,), (0,)), ((), ())),
            precision=jax.lax.Precision.HIGHEST,
            preferred_element_type=jnp.float32)
        cvx = mimg(g[:, 0:1], fx_i) * L
        cvy = mimg(g[:, 1:2], fy_i) * L
        cvz = mimg(g[:, 2:3], fz_i) * L
        nrm = jnp.sqrt(cvx * cvx + cvy * cvy + cvz * cvz) + 1e-12
        ux.append((cvx / nrm).astype(jnp.bfloat16).astype(jnp.float32))
        uy.append((cvy / nrm).astype(jnp.bfloat16).astype(jnp.float32))
        uz.append((cvz / nrm).astype(jnp.bfloat16).astype(jnp.float32))
        cur = jnp.where(hit, 1e9, cur)

    acc = jnp.zeros((BR, 1), jnp.float32)
    for a in range(4):
        for b in range(a + 1, 4):
            c = ux[a] * ux[b] + uy[a] * uy[b] + uz[a] * uz[b]
            acc = acc + (c + 1.0 / 3.0) ** 2
    qv = 1.0 - 0.375 * acc
    vrow = (s_i < 0.5) & (m3 < cutoff)
    qnum = jnp.where(vrow, qv, 0.0)
    qden = jnp.where(vrow, 1.0, 0.0)
    qout_ref[...] = jnp.concatenate(
        [qnum, qden, jnp.zeros((BR, 6), jnp.float32)], axis=1)


def _count_body(keys_hbm, out_hbm, buf, hist):
    wid = lax.axis_index("s") * NC + lax.axis_index("c")
    base = lax.iota(jnp.int32, NL) * NH
    ones = jnp.full((NL,), 1, jnp.int32)
    zer = jnp.zeros((NL,), jnp.int32)

    def zbody(j, _):
        hist[pl.ds(j * NL, NL)] = zer
        return 0
    lax.fori_loop(0, (NL * NH) // NL, zbody, 0)

    def sub(s, _):
        off = wid * CHUNK + s * SUB
        pltpu.sync_copy(keys_hbm.at[pl.ds(off, SUB)], buf)

        def inner(i, _):
            k16 = buf[pl.ds(i * NL, NL)]
            plsc.addupdate_scatter(hist, [base + k16], ones)
            return 0
        lax.fori_loop(0, SUB // NL, inner, 0)
        return 0
    lax.fori_loop(0, NSUB, sub, 0)

    pltpu.sync_copy(hist, out_hbm.at[pl.ds(wid * NL * NH, NL * NH)])


def _final_kernel(cnt_ref, rb_ref, qb_ref, qo_ref, sp_ref, cell_ref, par_ref,
                  g_ref, t_ref, s_ref, qt_ref):
    b0 = par_ref[0, 0]
    b1 = par_ref[0, 1]
    dr = par_ref[0, 2]

    c00 = cell_ref[0, 0]
    c01 = cell_ref[0, 1]
    c02 = cell_ref[0, 2]
    c10 = cell_ref[1, 0]
    c11 = cell_ref[1, 1]
    c12 = cell_ref[1, 2]
    c20 = cell_ref[2, 0]
    c21 = cell_ref[2, 1]
    c22 = cell_ref[2, 2]
    vol = jnp.abs(c00 * (c11 * c22 - c12 * c21)
                  - c01 * (c10 * c22 - c12 * c20)
                  + c02 * (c10 * c21 - c11 * c20))
    n_f = jnp.float32(N)
    rho = n_f / vol

    n1 = jnp.sum(sp_ref[...]).astype(jnp.float32)
    bbar = (b0 * (n_f - n1) + b1 * n1) / n_f
    bbar2 = bbar * bbar

    cnt = cnt_ref[...].astype(jnp.float32)
    tot = jnp.sum(cnt, axis=0, keepdims=True)
    c0 = tot[:, 0:NKEY]
    c1 = tot[:, NKEY:2 * NKEY]
    c2 = tot[:, 2 * NKEY:3 * NKEY]
    hist = b0 * b0 * c0 + b0 * b1 * c1 + b1 * b1 * c2

    rb = rb_ref[...]
    lane = lax.broadcasted_iota(jnp.int32, (1, NKEY), 1)
    live = lane < NBINS
    pi = jnp.float32(3.14159265358979323846)
    shell = 4.0 * pi * rb * rb * dr
    g = hist / (n_f * rho * jnp.where(live, shell, 1.0) * bbar2)
    g = jnp.where(live, g, 0.0)
    g_ref[...] = g
    t_ref[...] = 4.0 * pi * rb * rho * bbar2 * g

    q = qb_ref[...]
    integ = jnp.where(live, rb * (g - 1.0) * dr, 0.0)
    sq = jnp.sin(q * rb)
    s_ref[...] = 1.0 + (4.0 * pi * rho / q) * jnp.sum(
        sq * integ, axis=1, keepdims=True)

    qnum = jnp.sum(qo_ref[:, 0:1])
    qden = jnp.sum(qo_ref[:, 1:2])
    qt_ref[...] = jnp.full((1, 1), qnum / jnp.maximum(qden, 1.0), jnp.float32)


def _counts_sc(keys_flat):
    mesh = plsc.VectorSubcoreMesh(core_axis_name="c", subcore_axis_name="s")
    run = functools.partial(
        pl.kernel, mesh=mesh,
        compiler_params=pltpu.CompilerParams(needs_layout_passes=False),
        out_type=jax.ShapeDtypeStruct((NW * NL * NH,), jnp.int32),
        scratch_types=[
            pltpu.VMEM((SUB,), jnp.int32),
            pltpu.VMEM((NL * NH,), jnp.int32),
        ],
    )(_count_body)
    return run(keys_flat)


def kernel(positions, cell, species, r_bins, q_bins, b_coh):
    L = cell[0, 0]
    frac = positions @ jnp.linalg.inv(cell)
    fx, fy, fz = frac[:, 0], frac[:, 1], frac[:, 2]
    s_f = species.astype(jnp.float32)

    rowpack = jnp.stack([fx, fy, fz, s_f] + [jnp.zeros((N,), jnp.float32)] * 4,
                        axis=1)
    colpack = jnp.stack([fx, fy, fz, s_f] + [jnp.zeros((N,), jnp.float32)] * 4,
                        axis=0)
    r0 = r_bins[0]
    dr = r_bins[1] - r_bins[0]
    parA = jnp.stack([L, r0, dr, r_bins[-1], jnp.float32(3.5),
                      jnp.float32(0), jnp.float32(0), jnp.float32(0)]
                     ).reshape(1, 8)

    keys, qout = pl.pallas_call(
        _pairs_kernel,
        grid=(N // BR,),
        in_specs=[
            pl.BlockSpec((BR, 8), lambda i: (i, 0)),
            pl.BlockSpec((8, N), lambda i: (0, 0)),
            pl.BlockSpec((N, 8), lambda i: (0, 0)),
            pl.BlockSpec((1, 8), lambda i: (0, 0)),
        ],
        out_specs=[
            pl.BlockSpec((BR, N), lambda i: (i, 0)),
            pl.BlockSpec((BR, 8), lambda i: (i, 0)),
        ],
        out_shape=[
            jax.ShapeDtypeStruct((N, N), jnp.int32),
            jax.ShapeDtypeStruct((N, 8), jnp.float32),
        ],
    )(rowpack, colpack, rowpack, parA)

    counts = _counts_sc(keys.reshape(TOTAL))

    rb_pad = jnp.concatenate(
        [r_bins, jnp.zeros((NKEY - NBINS,), jnp.float32)]).reshape(1, NKEY)
    parC = jnp.stack([b_coh[0], b_coh[1], dr,
                      jnp.float32(0), jnp.float32(0), jnp.float32(0),
                      jnp.float32(0), jnp.float32(0)]).reshape(1, 8)

    g_p, t_p, s_p, qt = pl.pallas_call(
        _final_kernel,
        out_shape=[
            jax.ShapeDtypeStruct((1, NKEY), jnp.float32),
            jax.ShapeDtypeStruct((1, NKEY), jnp.float32),
            jax.ShapeDtypeStruct((NQ, 1), jnp.float32),
            jax.ShapeDtypeStruct((1, 1), jnp.float32),
        ],
    )(counts.reshape(NW * NL, NH), rb_pad, q_bins.reshape(NQ, 1),
      qout, species.reshape(8, N // 8), cell, parC)

    return g_p[0, :NBINS], t_p[0, :NBINS], s_p[:, 0], qt[0, 0]


# speedup vs baseline: 27.6247x; 1.0491x over previous
import functools

import jax
import jax.numpy as jnp
from jax import lax
from jax.experimental import pallas as pl
from jax.experimental.pallas import tpu as pltpu
from jax.experimental.pallas import tpu_sc as plsc

N = 4096
NBINS = 500
NQ = 400
BR = 128
NKEY = 512
NH = 3 * NKEY + 16
DEAD = 3 * NKEY

NC, NS, NL = 2, 16, 16
NW = NC * NS
TOTAL = N * N
CHUNK = TOTAL // NW
SUB = 32768
NSUB = CHUNK // SUB


def _pairs_kernel(row_ref, col_ref, tab_ref, par_ref, keys_ref, qout_ref):
    L = par_ref[0, 0].astype(jnp.bfloat16).astype(jnp.float32)
    r0 = par_ref[0, 1]
    dr = par_ref[0, 2]
    rlast = par_ref[0, 3]
    cutoff = par_ref[0, 4]

    fx_i = row_ref[:, 0:1]
    fy_i = row_ref[:, 1:2]
    fz_i = row_ref[:, 2:3]
    s_i = row_ref[:, 3:4]
    fx_j = col_ref[0:1, :]
    fy_j = col_ref[1:2, :]
    fz_j = col_ref[2:3, :]
    s_j = col_ref[3:4, :]

    def mimg(a, b):
        d = a - b
        d = d - jnp.round(d)
        return d.astype(jnp.bfloat16).astype(jnp.float32)

    dx = mimg(fx_i, fx_j) * L
    dy = mimg(fy_i, fy_j) * L
    dz = mimg(fz_i, fz_j) * L
    r = jnp.sqrt(dx * dx + dy * dy + dz * dz + 1e-12)

    idx = jnp.clip(jnp.floor((r - r0) / dr).astype(jnp.int32), 0, NBINS - 1)
    valid = (r >= r0) & (r < rlast + dr)
    cls = (s_i + s_j).astype(jnp.int32)
    key = idx + cls * NKEY
    keys_ref[...] = jnp.where(valid, key, DEAD)

    row_g = pl.program_id(0) * BR + lax.broadcasted_iota(jnp.int32, (BR, N), 0)
    col_g = lax.broadcasted_iota(jnp.int32, (BR, N), 1)
    eye = row_g == col_g
    cur = r + jnp.where(eye | (s_j < 0.5), 1e6, 0.0)

    ux, uy, uz = [], [], []
    m3 = None
    for _ in range(4):
        mk = jnp.min(cur, axis=1, keepdims=True)
        m3 = mk
        jstar = jnp.min(jnp.where(cur == mk, col_g, N), axis=1, keepdims=True)
        hit = col_g == jstar
        oh = hit.astype(jnp.float32)
        g = jax.lax.dot_general(
            oh, tab_ref[...], (((1,), (0,)), ((), ())),
            precision=jax.lax.Precision.HIGHEST,
            preferred_element_type=jnp.float32)
        cvx = mimg(g[:, 0:1], fx_i) * L
        cvy = mimg(g[:, 1:2], fy_i) * L
        cvz = mimg(g[:, 2:3], fz_i) * L
        nrm = jnp.sqrt(cvx * cvx + cvy * cvy + cvz * cvz) + 1e-12
        ux.append((cvx / nrm).astype(jnp.bfloat16).astype(jnp.float32))
        uy.append((cvy / nrm).astype(jnp.bfloat16).astype(jnp.float32))
        uz.append((cvz / nrm).astype(jnp.bfloat16).astype(jnp.float32))
        cur = jnp.where(hit, 1e9, cur)

    acc = jnp.zeros((BR, 1), jnp.float32)
    for a in range(4):
        for b in range(a + 1, 4):
            c = ux[a] * ux[b] + uy[a] * uy[b] + uz[a] * uz[b]
            acc = acc + (c + 1.0 / 3.0) ** 2
    qv = 1.0 - 0.375 * acc
    vrow = (s_i < 0.5) & (m3 < cutoff)
    qnum = jnp.where(vrow, qv, 0.0)
    qden = jnp.where(vrow, 1.0, 0.0)
    qout_ref[...] = jnp.concatenate(
        [qnum, qden, jnp.zeros((BR, 6), jnp.float32)], axis=1)


def _count_body(keys_hbm, out_hbm, buf, hist):
    nrows = keys_hbm.shape[0]
    rows_w = nrows // NW
    nsub = rows_w // 8
    wid = lax.axis_index("s") * NC + lax.axis_index("c")
    base = lax.iota(jnp.int32, NL) * NH
    ones = jnp.full((NL,), 1, jnp.int32)
    zer = jnp.zeros((NL,), jnp.int32)

    def zbody(j, _):
        hist[pl.ds(j * NL, NL)] = zer
        return 0
    lax.fori_loop(0, (NL * NH) // NL, zbody, 0)

    def sub(s, _):
        row0 = wid * rows_w + s * 8
        pltpu.sync_copy(keys_hbm.at[pl.ds(row0, 8), :], buf)

        def inner(i, _):
            for rr in range(8):
                k16 = buf[rr, pl.ds(i * NL, NL)]
                plsc.addupdate_scatter(hist, [base + k16], ones)
            return 0
        lax.fori_loop(0, N // NL, inner, 0)
        return 0
    lax.fori_loop(0, nsub, sub, 0)

    pltpu.sync_copy(hist, out_hbm.at[pl.ds(wid * NL * NH, NL * NH)])


def _final_kernel(cnt_ref, rb_ref, qb_ref, qo_ref, sp_ref, cell_ref, par_ref,
                  g_ref, t_ref, s_ref, qt_ref):
    b0 = par_ref[0, 0]
    b1 = par_ref[0, 1]
    dr = par_ref[0, 2]

    c00 = cell_ref[0, 0]
    c01 = cell_ref[0, 1]
    c02 = cell_ref[0, 2]
    c10 = cell_ref[1, 0]
    c11 = cell_ref[1, 1]
    c12 = cell_ref[1, 2]
    c20 = cell_ref[2, 0]
    c21 = cell_ref[2, 1]
    c22 = cell_ref[2, 2]
    vol = jnp.abs(c00 * (c11 * c22 - c12 * c21)
                  - c01 * (c10 * c22 - c12 * c20)
                  + c02 * (c10 * c21 - c11 * c20))
    n_f = jnp.float32(N)
    rho = n_f / vol

    n1 = jnp.sum(sp_ref[...]).astype(jnp.float32)
    bbar = (b0 * (n_f - n1) + b1 * n1) / n_f
    bbar2 = bbar * bbar

    cnt = cnt_ref[...].astype(jnp.float32)
    tot = jnp.sum(cnt, axis=0, keepdims=True)
    c0 = tot[:, 0:NKEY]
    c1 = tot[:, NKEY:2 * NKEY]
    c2 = tot[:, 2 * NKEY:3 * NKEY]
    hist = b0 * b0 * c0 + b0 * b1 * c1 + b1 * b1 * c2

    rb = rb_ref[...]
    lane = lax.broadcasted_iota(jnp.int32, (1, NKEY), 1)
    live = lane < NBINS
    pi = jnp.float32(3.14159265358979323846)
    shell = 4.0 * pi * rb * rb * dr
    g = hist / (n_f * rho * jnp.where(live, shell, 1.0) * bbar2)
    g = jnp.where(live, g, 0.0)
    g_ref[...] = g
    t_ref[...] = 4.0 * pi * rb * rho * bbar2 * g

    q = qb_ref[...]
    integ = jnp.where(live, rb * (g - 1.0) * dr, 0.0)
    sq = jnp.sin(q * rb)
    s_ref[...] = 1.0 + (4.0 * pi * rho / q) * jnp.sum(
        sq * integ, axis=1, keepdims=True)

    qnum = jnp.sum(qo_ref[:, 0:1])
    qden = jnp.sum(qo_ref[:, 1:2])
    qt_ref[...] = jnp.full((1, 1), qnum / jnp.maximum(qden, 1.0), jnp.float32)


def _counts_sc(keys2d):
    mesh = plsc.VectorSubcoreMesh(core_axis_name="c", subcore_axis_name="s")
    run = functools.partial(
        pl.kernel, mesh=mesh,
        compiler_params=pltpu.CompilerParams(needs_layout_passes=False),
        out_type=jax.ShapeDtypeStruct((NW * NL * NH,), jnp.int32),
        scratch_types=[
            pltpu.VMEM((8, N), jnp.int32),
            pltpu.VMEM((NL * NH,), jnp.int32),
        ],
    )(_count_body)
    return run(keys2d)


def kernel(positions, cell, species, r_bins, q_bins, b_coh):
    L = cell[0, 0]
    frac = positions @ jnp.linalg.inv(cell)
    fx, fy, fz = frac[:, 0], frac[:, 1], frac[:, 2]
    s_f = species.astype(jnp.float32)

    rowpack = jnp.stack([fx, fy, fz, s_f] + [jnp.zeros((N,), jnp.float32)] * 4,
                        axis=1)
    colpack = jnp.stack([fx, fy, fz, s_f] + [jnp.zeros((N,), jnp.float32)] * 4,
                        axis=0)
    r0 = r_bins[0]
    dr = r_bins[1] - r_bins[0]
    parA = jnp.stack([L, r0, dr, r_bins[-1], jnp.float32(3.5),
                      jnp.float32(0), jnp.float32(0), jnp.float32(0)]
                     ).reshape(1, 8)

    keys, qout = pl.pallas_call(
        _pairs_kernel,
        grid=(N // BR,),
        in_specs=[
            pl.BlockSpec((BR, 8), lambda i: (i, 0)),
            pl.BlockSpec((8, N), lambda i: (0, 0)),
            pl.BlockSpec((N, 8), lambda i: (0, 0)),
            pl.BlockSpec((1, 8), lambda i: (0, 0)),
        ],
        out_specs=[
            pl.BlockSpec((BR, N), lambda i: (i, 0)),
            pl.BlockSpec((BR, 8), lambda i: (i, 0)),
        ],
        out_shape=[
            jax.ShapeDtypeStruct((N, N), jnp.int32),
            jax.ShapeDtypeStruct((N, 8), jnp.float32),
        ],
    )(rowpack, colpack, rowpack, parA)

    counts = _counts_sc(keys)

    rb_pad = jnp.concatenate(
        [r_bins, jnp.zeros((NKEY - NBINS,), jnp.float32)]).reshape(1, NKEY)
    parC = jnp.stack([b_coh[0], b_coh[1], dr,
                      jnp.float32(0), jnp.float32(0), jnp.float32(0),
                      jnp.float32(0), jnp.float32(0)]).reshape(1, 8)

    g_p, t_p, s_p, qt = pl.pallas_call(
        _final_kernel,
        out_shape=[
            jax.ShapeDtypeStruct((1, NKEY), jnp.float32),
            jax.ShapeDtypeStruct((1, NKEY), jnp.float32),
            jax.ShapeDtypeStruct((NQ, 1), jnp.float32),
            jax.ShapeDtypeStruct((1, 1), jnp.float32),
        ],
    )(counts.reshape(NW * NL, NH), rb_pad, q_bins.reshape(NQ, 1),
      qout, species.reshape(8, N // 8), cell, parC)

    return g_p[0, :NBINS], t_p[0, :NBINS], s_p[:, 0], qt[0, 0]


# speedup vs baseline: 30.5027x; 1.1042x over previous
import functools

import jax
import jax.numpy as jnp
from jax import lax
from jax.experimental import pallas as pl
from jax.experimental.pallas import tpu as pltpu
from jax.experimental.pallas import tpu_sc as plsc

N = 4096
NBINS = 500
NQ = 400
BR = 128
NKEY = 512
NH = 3 * NKEY + 16
DEAD = 3 * NKEY

NC, NS, NL = 2, 16, 16
NW = NC * NS
TOTAL = N * N
CHUNK = TOTAL // NW
SUB = 32768
NSUB = CHUNK // SUB


def _pairs_kernel(row_ref, col_ref, tab_ref, par_ref, keys_ref, qout_ref):
    L = par_ref[0, 0].astype(jnp.bfloat16).astype(jnp.float32)
    r0 = par_ref[0, 1]
    dr = par_ref[0, 2]
    rlast = par_ref[0, 3]
    cutoff = par_ref[0, 4]

    fx_i = row_ref[:, 0:1]
    fy_i = row_ref[:, 1:2]
    fz_i = row_ref[:, 2:3]
    s_i = row_ref[:, 3:4]
    fx_j = col_ref[0:1, :]
    fy_j = col_ref[1:2, :]
    fz_j = col_ref[2:3, :]
    s_j = col_ref[3:4, :]

    def mimg(a, b):
        d = a - b
        d = d - jnp.round(d)
        return d.astype(jnp.bfloat16).astype(jnp.float32)

    dx = mimg(fx_i, fx_j) * L
    dy = mimg(fy_i, fy_j) * L
    dz = mimg(fz_i, fz_j) * L
    r = jnp.sqrt(dx * dx + dy * dy + dz * dz + 1e-12)

    row_g = pl.program_id(0) * BR + lax.broadcasted_iota(jnp.int32, (BR, N), 0)
    col_g = lax.broadcasted_iota(jnp.int32, (BR, N), 1)
    eye = row_g == col_g

    idx = jnp.clip(jnp.floor((r - r0) / dr).astype(jnp.int32), 0, NBINS - 1)
    valid = (r >= r0) & (r < rlast + dr) & (col_g > row_g)
    cls = (s_i + s_j).astype(jnp.int32)
    key = idx + cls * NKEY
    keys_ref[...] = jnp.where(valid, key, DEAD)

    cur = r + jnp.where(eye | (s_j < 0.5), 1e6, 0.0)

    ux, uy, uz = [], [], []
    m3 = None
    for _ in range(4):
        mk = jnp.min(cur, axis=1, keepdims=True)
        m3 = mk
        jstar = jnp.min(jnp.where(cur == mk, col_g, N), axis=1, keepdims=True)
        hit = col_g == jstar
        oh = hit.astype(jnp.float32)
        g = jax.lax.dot_general(
            oh, tab_ref[...], (((1,), (0,)), ((), ())),
            precision=jax.lax.Precision.HIGHEST,
            preferred_element_type=jnp.float32)
        cvx = mimg(g[:, 0:1], fx_i) * L
        cvy = mimg(g[:, 1:2], fy_i) * L
        cvz = mimg(g[:, 2:3], fz_i) * L
        nrm = jnp.sqrt(cvx * cvx + cvy * cvy + cvz * cvz) + 1e-12
        ux.append((cvx / nrm).astype(jnp.bfloat16).astype(jnp.float32))
        uy.append((cvy / nrm).astype(jnp.bfloat16).astype(jnp.float32))
        uz.append((cvz / nrm).astype(jnp.bfloat16).astype(jnp.float32))
        cur = jnp.where(hit, 1e9, cur)

    acc = jnp.zeros((BR, 1), jnp.float32)
    for a in range(4):
        for b in range(a + 1, 4):
            c = ux[a] * ux[b] + uy[a] * uy[b] + uz[a] * uz[b]
            acc = acc + (c + 1.0 / 3.0) ** 2
    qv = 1.0 - 0.375 * acc
    vrow = (s_i < 0.5) & (m3 < cutoff)
    qnum = jnp.where(vrow, qv, 0.0)
    qden = jnp.where(vrow, 1.0, 0.0)
    qout_ref[...] = jnp.concatenate(
        [qnum, qden, jnp.zeros((BR, 6), jnp.float32)], axis=1)


def _count_body(keys_hbm, out_hbm, buf, hist):
    nrows = keys_hbm.shape[0]
    rows_w = nrows // NW
    nsub = rows_w // 8
    wid = lax.axis_index("s") * NC + lax.axis_index("c")
    base = lax.iota(jnp.int32, NL) * NH
    ones = jnp.full((NL,), 1, jnp.int32)
    zer = jnp.zeros((NL,), jnp.int32)

    def zbody(j, _):
        hist[pl.ds(j * NL, NL)] = zer
        return 0
    lax.fori_loop(0, (NL * NH) // NL, zbody, 0)

    def sub(s, _):
        row0 = wid * rows_w + s * 8
        pltpu.sync_copy(keys_hbm.at[pl.ds(row0, 8), :], buf)

        def inner(i, _):
            for rr in range(8):
                k16 = buf[rr, pl.ds(i * NL, NL)]
                plsc.addupdate_scatter(hist, [base + k16], ones,
                                       mask=k16 < DEAD)
            return 0
        lax.fori_loop(0, N // NL, inner, 0)
        return 0
    lax.fori_loop(0, nsub, sub, 0)

    pltpu.sync_copy(hist, out_hbm.at[pl.ds(wid * NL * NH, NL * NH)])


def _final_kernel(cnt_ref, rb_ref, qb_ref, qo_ref, sp_ref, cell_ref, par_ref,
                  g_ref, t_ref, s_ref, qt_ref):
    b0 = par_ref[0, 0]
    b1 = par_ref[0, 1]
    dr = par_ref[0, 2]

    c00 = cell_ref[0, 0]
    c01 = cell_ref[0, 1]
    c02 = cell_ref[0, 2]
    c10 = cell_ref[1, 0]
    c11 = cell_ref[1, 1]
    c12 = cell_ref[1, 2]
    c20 = cell_ref[2, 0]
    c21 = cell_ref[2, 1]
    c22 = cell_ref[2, 2]
    vol = jnp.abs(c00 * (c11 * c22 - c12 * c21)
                  - c01 * (c10 * c22 - c12 * c20)
                  + c02 * (c10 * c21 - c11 * c20))
    n_f = jnp.float32(N)
    rho = n_f / vol

    n1 = jnp.sum(sp_ref[...]).astype(jnp.float32)
    bbar = (b0 * (n_f - n1) + b1 * n1) / n_f
    bbar2 = bbar * bbar

    cnt = cnt_ref[...].astype(jnp.float32)
    tot = jnp.sum(cnt, axis=0, keepdims=True)
    c0 = tot[:, 0:NKEY]
    c1 = tot[:, NKEY:2 * NKEY]
    c2 = tot[:, 2 * NKEY:3 * NKEY]
    hist = 2.0 * (b0 * b0 * c0 + b0 * b1 * c1 + b1 * b1 * c2)

    rb = rb_ref[...]
    lane = lax.broadcasted_iota(jnp.int32, (1, NKEY), 1)
    live = lane < NBINS
    pi = jnp.float32(3.14159265358979323846)
    shell = 4.0 * pi * rb * rb * dr
    g = hist / (n_f * rho * jnp.where(live, shell, 1.0) * bbar2)
    g = jnp.where(live, g, 0.0)
    g_ref[...] = g
    t_ref[...] = 4.0 * pi * rb * rho * bbar2 * g

    q = qb_ref[...]
    integ = jnp.where(live, rb * (g - 1.0) * dr, 0.0)
    sq = jnp.sin(q * rb)
    s_ref[...] = 1.0 + (4.0 * pi * rho / q) * jnp.sum(
        sq * integ, axis=1, keepdims=True)

    qnum = jnp.sum(qo_ref[:, 0:1])
    qden = jnp.sum(qo_ref[:, 1:2])
    qt_ref[...] = jnp.full((1, 1), qnum / jnp.maximum(qden, 1.0), jnp.float32)


def _counts_sc(keys2d):
    mesh = plsc.VectorSubcoreMesh(core_axis_name="c", subcore_axis_name="s")
    run = functools.partial(
        pl.kernel, mesh=mesh,
        compiler_params=pltpu.CompilerParams(needs_layout_passes=False),
        out_type=jax.ShapeDtypeStruct((NW * NL * NH,), jnp.int32),
        scratch_types=[
            pltpu.VMEM((8, N), jnp.int32),
            pltpu.VMEM((NL * NH,), jnp.int32),
        ],
    )(_count_body)
    return run(keys2d)


def kernel(positions, cell, species, r_bins, q_bins, b_coh):
    L = cell[0, 0]
    frac = positions @ jnp.linalg.inv(cell)
    fx, fy, fz = frac[:, 0], frac[:, 1], frac[:, 2]
    s_f = species.astype(jnp.float32)

    rowpack = jnp.stack([fx, fy, fz, s_f] + [jnp.zeros((N,), jnp.float32)] * 4,
                        axis=1)
    colpack = jnp.stack([fx, fy, fz, s_f] + [jnp.zeros((N,), jnp.float32)] * 4,
                        axis=0)
    r0 = r_bins[0]
    dr = r_bins[1] - r_bins[0]
    parA = jnp.stack([L, r0, dr, r_bins[-1], jnp.float32(3.5),
                      jnp.float32(0), jnp.float32(0), jnp.float32(0)]
                     ).reshape(1, 8)

    keys, qout = pl.pallas_call(
        _pairs_kernel,
        grid=(N // BR,),
        in_specs=[
            pl.BlockSpec((BR, 8), lambda i: (i, 0)),
            pl.BlockSpec((8, N), lambda i: (0, 0)),
            pl.BlockSpec((N, 8), lambda i: (0, 0)),
            pl.BlockSpec((1, 8), lambda i: (0, 0)),
        ],
        out_specs=[
            pl.BlockSpec((BR, N), lambda i: (i, 0)),
            pl.BlockSpec((BR, 8), lambda i: (i, 0)),
        ],
        out_shape=[
            jax.ShapeDtypeStruct((N, N), jnp.int32),
            jax.ShapeDtypeStruct((N, 8), jnp.float32),
        ],
    )(rowpack, colpack, rowpack, parA)

    counts = _counts_sc(keys)

    rb_pad = jnp.concatenate(
        [r_bins, jnp.zeros((NKEY - NBINS,), jnp.float32)]).reshape(1, NKEY)
    parC = jnp.stack([b_coh[0], b_coh[1], dr,
                      jnp.float32(0), jnp.float32(0), jnp.float32(0),
                      jnp.float32(0), jnp.float32(0)]).reshape(1, 8)

    g_p, t_p, s_p, qt = pl.pallas_call(
        _final_kernel,
        out_shape=[
            jax.ShapeDtypeStruct((1, NKEY), jnp.float32),
            jax.ShapeDtypeStruct((1, NKEY), jnp.float32),
            jax.ShapeDtypeStruct((NQ, 1), jnp.float32),
            jax.ShapeDtypeStruct((1, 1), jnp.float32),
        ],
    )(counts.reshape(NW * NL, NH), rb_pad, q_bins.reshape(NQ, 1),
      qout, species.reshape(8, N // 8), cell, parC)

    return g_p[0, :NBINS], t_p[0, :NBINS], s_p[:, 0], qt[0, 0]


# speedup vs baseline: 39.8041x; 1.3049x over previous
import functools

import jax
import jax.numpy as jnp
from jax import lax
from jax.experimental import pallas as pl
from jax.experimental.pallas import tpu as pltpu
from jax.experimental.pallas import tpu_sc as plsc

N = 4096
NBINS = 500
NQ = 400
BR = 128
NKEY = 512
NH = 3 * NKEY + 16
DEAD = 3 * NKEY

NC, NS, NL = 2, 16, 16
NW = NC * NS
TOTAL = N * N
CHUNK = TOTAL // NW
SUB = 32768
NSUB = CHUNK // SUB


def _pairs_kernel(row_ref, col_ref, tab_ref, par_ref, keys_ref, qout_ref):
    L = par_ref[0, 0].astype(jnp.bfloat16).astype(jnp.float32)
    r0 = par_ref[0, 1]
    dr = par_ref[0, 2]
    rlast = par_ref[0, 3]
    cutoff = par_ref[0, 4]

    fx_i = row_ref[:, 0:1]
    fy_i = row_ref[:, 1:2]
    fz_i = row_ref[:, 2:3]
    s_i = row_ref[:, 3:4]
    fx_j = col_ref[0:1, :]
    fy_j = col_ref[1:2, :]
    fz_j = col_ref[2:3, :]
    s_j = col_ref[3:4, :]

    def mimg(a, b):
        d = a - b
        d = d - jnp.round(d)
        return d.astype(jnp.bfloat16).astype(jnp.float32)

    dx = mimg(fx_i, fx_j) * L
    dy = mimg(fy_i, fy_j) * L
    dz = mimg(fz_i, fz_j) * L
    r = jnp.sqrt(dx * dx + dy * dy + dz * dz + 1e-12)

    rowoff = par_ref[0, 5].astype(jnp.int32)
    row_g = (rowoff + pl.program_id(0) * BR
             + lax.broadcasted_iota(jnp.int32, (BR, N), 0))
    col_g = lax.broadcasted_iota(jnp.int32, (BR, N), 1)
    eye = row_g == col_g

    idx = jnp.clip(jnp.floor((r - r0) / dr).astype(jnp.int32), 0, NBINS - 1)
    valid = (r >= r0) & (r < rlast + dr) & (col_g > row_g)
    cls = (s_i + s_j).astype(jnp.int32)
    key = idx + cls * NKEY
    keys_ref[...] = jnp.where(valid, key, DEAD)

    cur = r + jnp.where(eye | (s_j < 0.5), 1e6, 0.0)

    ux, uy, uz = [], [], []
    m3 = None
    for _ in range(4):
        mk = jnp.min(cur, axis=1, keepdims=True)
        m3 = mk
        jstar = jnp.min(jnp.where(cur == mk, col_g, N), axis=1, keepdims=True)
        hit = col_g == jstar
        oh = hit.astype(jnp.float32)
        g = jax.lax.dot_general(
            oh, tab_ref[...], (((1,), (0,)), ((), ())),
            precision=jax.lax.Precision.HIGHEST,
            preferred_element_type=jnp.float32)
        cvx = mimg(g[:, 0:1], fx_i) * L
        cvy = mimg(g[:, 1:2], fy_i) * L
        cvz = mimg(g[:, 2:3], fz_i) * L
        nrm = jnp.sqrt(cvx * cvx + cvy * cvy + cvz * cvz) + 1e-12
        ux.append((cvx / nrm).astype(jnp.bfloat16).astype(jnp.float32))
        uy.append((cvy / nrm).astype(jnp.bfloat16).astype(jnp.float32))
        uz.append((cvz / nrm).astype(jnp.bfloat16).astype(jnp.float32))
        cur = jnp.where(hit, 1e9, cur)

    acc = jnp.zeros((BR, 1), jnp.float32)
    for a in range(4):
        for b in range(a + 1, 4):
            c = ux[a] * ux[b] + uy[a] * uy[b] + uz[a] * uz[b]
            acc = acc + (c + 1.0 / 3.0) ** 2
    qv = 1.0 - 0.375 * acc
    vrow = (s_i < 0.5) & (m3 < cutoff)
    qnum = jnp.where(vrow, qv, 0.0)
    qden = jnp.where(vrow, 1.0, 0.0)
    qout_ref[...] = jnp.concatenate(
        [qnum, qden, jnp.zeros((BR, 6), jnp.float32)], axis=1)


def _count_body(keys_hbm, out_hbm, buf, hist):
    nrows = keys_hbm.shape[0]
    rows_w = nrows // NW
    nsub = rows_w // 8
    wid = lax.axis_index("s") * NC + lax.axis_index("c")
    base = lax.iota(jnp.int32, NL) * NH
    ones = jnp.full((NL,), 1, jnp.int32)
    zer = jnp.zeros((NL,), jnp.int32)

    def zbody(j, _):
        hist[pl.ds(j * NL, NL)] = zer
        return 0
    lax.fori_loop(0, (NL * NH) // NL, zbody, 0)

    def sub(s, _):
        row0 = wid * rows_w + s * 8
        pltpu.sync_copy(keys_hbm.at[pl.ds(row0, 8), :], buf)

        def inner(i, _):
            for rr in range(8):
                k16 = buf[rr, pl.ds(i * NL, NL)]
                plsc.addupdate_scatter(hist, [base + k16], ones,
                                       mask=k16 < DEAD)
            return 0
        lax.fori_loop(0, N // NL, inner, 0)
        return 0
    lax.fori_loop(0, nsub, sub, 0)

    pltpu.sync_copy(hist, out_hbm.at[pl.ds(wid * NL * NH, NL * NH)])


def _final_kernel(cn0_ref, cn1_ref, cn2_ref, cn3_ref, rb_ref, qb_ref,
                  qo0_ref, qo1_ref, qo2_ref, qo3_ref, sp_ref, cell_ref,
                  par_ref, g_ref, t_ref, s_ref, qt_ref):
    b0 = par_ref[0, 0]
    b1 = par_ref[0, 1]
    dr = par_ref[0, 2]

    c00 = cell_ref[0, 0]
    c01 = cell_ref[0, 1]
    c02 = cell_ref[0, 2]
    c10 = cell_ref[1, 0]
    c11 = cell_ref[1, 1]
    c12 = cell_ref[1, 2]
    c20 = cell_ref[2, 0]
    c21 = cell_ref[2, 1]
    c22 = cell_ref[2, 2]
    vol = jnp.abs(c00 * (c11 * c22 - c12 * c21)
                  - c01 * (c10 * c22 - c12 * c20)
                  + c02 * (c10 * c21 - c11 * c20))
    n_f = jnp.float32(N)
    rho = n_f / vol

    n1 = jnp.sum(sp_ref[...]).astype(jnp.float32)
    bbar = (b0 * (n_f - n1) + b1 * n1) / n_f
    bbar2 = bbar * bbar

    cnt = (cn0_ref[...] + cn1_ref[...] + cn2_ref[...] + cn3_ref[...]
           ).astype(jnp.float32)
    tot = jnp.sum(cnt, axis=0, keepdims=True)
    c0 = tot[:, 0:NKEY]
    c1 = tot[:, NKEY:2 * NKEY]
    c2 = tot[:, 2 * NKEY:3 * NKEY]
    hist = 2.0 * (b0 * b0 * c0 + b0 * b1 * c1 + b1 * b1 * c2)

    rb = rb_ref[...]
    lane = lax.broadcasted_iota(jnp.int32, (1, NKEY), 1)
    live = lane < NBINS
    pi = jnp.float32(3.14159265358979323846)
    shell = 4.0 * pi * rb * rb * dr
    g = hist / (n_f * rho * jnp.where(live, shell, 1.0) * bbar2)
    g = jnp.where(live, g, 0.0)
    g_ref[...] = g
    t_ref[...] = 4.0 * pi * rb * rho * bbar2 * g

    q = qb_ref[...]
    integ = jnp.where(live, rb * (g - 1.0) * dr, 0.0)
    sq = jnp.sin(q * rb)
    s_ref[...] = 1.0 + (4.0 * pi * rho / q) * jnp.sum(
        sq * integ, axis=1, keepdims=True)

    qnum = (jnp.sum(qo0_ref[:, 0:1]) + jnp.sum(qo1_ref[:, 0:1])
            + jnp.sum(qo2_ref[:, 0:1]) + jnp.sum(qo3_ref[:, 0:1]))
    qden = (jnp.sum(qo0_ref[:, 1:2]) + jnp.sum(qo1_ref[:, 1:2])
            + jnp.sum(qo2_ref[:, 1:2]) + jnp.sum(qo3_ref[:, 1:2]))
    qt_ref[...] = jnp.full((1, 1), qnum / jnp.maximum(qden, 1.0), jnp.float32)


def _counts_sc(keys2d):
    mesh = plsc.VectorSubcoreMesh(core_axis_name="c", subcore_axis_name="s")
    run = functools.partial(
        pl.kernel, mesh=mesh,
        compiler_params=pltpu.CompilerParams(needs_layout_passes=False),
        out_type=jax.ShapeDtypeStruct((NW * NL * NH,), jnp.int32),
        scratch_types=[
            pltpu.VMEM((8, N), jnp.int32),
            pltpu.VMEM((NL * NH,), jnp.int32),
        ],
    )(_count_body)
    return run(keys2d)


def kernel(positions, cell, species, r_bins, q_bins, b_coh):
    L = cell[0, 0]
    frac = positions @ jnp.linalg.inv(cell)
    fx, fy, fz = frac[:, 0], frac[:, 1], frac[:, 2]
    s_f = species.astype(jnp.float32)

    rowpack = jnp.stack([fx, fy, fz, s_f] + [jnp.zeros((N,), jnp.float32)] * 4,
                        axis=1)
    colpack = jnp.stack([fx, fy, fz, s_f] + [jnp.zeros((N,), jnp.float32)] * 4,
                        axis=0)
    r0 = r_bins[0]
    dr = r_bins[1] - r_bins[0]

    NQROWS = N // 4
    counts_q = []
    qout_q = []
    for q in range(4):
        parA = jnp.stack([L, r0, dr, r_bins[-1], jnp.float32(3.5),
                          jnp.float32(q * NQROWS), jnp.float32(0),
                          jnp.float32(0)]).reshape(1, 8)
        keys, qout = pl.pallas_call(
            _pairs_kernel,
            grid=(NQROWS // BR,),
            in_specs=[
                pl.BlockSpec((BR, 8), lambda i: (i, 0)),
                pl.BlockSpec((8, N), lambda i: (0, 0)),
                pl.BlockSpec((N, 8), lambda i: (0, 0)),
                pl.BlockSpec((1, 8), lambda i: (0, 0)),
            ],
            out_specs=[
                pl.BlockSpec((BR, N), lambda i: (i, 0)),
                pl.BlockSpec((BR, 8), lambda i: (i, 0)),
            ],
            out_shape=[
                jax.ShapeDtypeStruct((NQROWS, N), jnp.int32),
                jax.ShapeDtypeStruct((NQROWS, 8), jnp.float32),
            ],
        )(rowpack[q * NQROWS:(q + 1) * NQROWS], colpack, rowpack, parA)
        counts_q.append(_counts_sc(keys))
        qout_q.append(qout)

    rb_pad = jnp.concatenate(
        [r_bins, jnp.zeros((NKEY - NBINS,), jnp.float32)]).reshape(1, NKEY)
    parC = jnp.stack([b_coh[0], b_coh[1], dr,
                      jnp.float32(0), jnp.float32(0), jnp.float32(0),
                      jnp.float32(0), jnp.float32(0)]).reshape(1, 8)

    g_p, t_p, s_p, qt = pl.pallas_call(
        _final_kernel,
        out_shape=[
            jax.ShapeDtypeStruct((1, NKEY), jnp.float32),
            jax.ShapeDtypeStruct((1, NKEY), jnp.float32),
            jax.ShapeDtypeStruct((NQ, 1), jnp.float32),
            jax.ShapeDtypeStruct((1, 1), jnp.float32),
        ],
    )(*[c.reshape(NW * NL, NH) for c in counts_q], rb_pad,
      q_bins.reshape(NQ, 1), *qout_q, species.reshape(8, N // 8), cell, parC)

    return g_p[0, :NBINS], t_p[0, :NBINS], s_p[:, 0], qt[0, 0]
